# baseline (speedup 1.0000x reference)
import functools

import jax
import jax.numpy as jnp
from jax import lax
from jax.experimental import pallas as pl
from jax.experimental.pallas import tpu as pltpu
from jax.experimental.pallas import tpu_sc as plsc

_NUM_CORES = 2
_NUM_SUBCORES = 16
_NW = _NUM_CORES * _NUM_SUBCORES
_G = 128
_NG = 3
_NO = 2
_NP = 3
_L = 16

_CPARAMS = pltpu.CompilerParams(needs_layout_passes=False)


def _make_pack(V, D):
    n_full = V // 128
    rem = V - n_full * 128
    mesh = plsc.VectorSubcoreMesh(core_axis_name="c", subcore_axis_name="s")
    kmax = (n_full + _NW - 1) // _NW
    n_steps = kmax + _NP

    @functools.partial(
        pl.kernel,
        mesh=mesh,
        out_type=jax.ShapeDtypeStruct((V // 2, 2 * D), jnp.float32),
        scratch_types=[
            pltpu.VMEM((_NP, D, 2 * D), jnp.float32),
            pltpu.VMEM((_NP, D, 2 * D), jnp.float32),
            pltpu.SemaphoreType.DMA((_NP,)),
            pltpu.SemaphoreType.DMA((_NP,)),
        ],
        compiler_params=_CPARAMS,
    )
    def body(wt_hbm, tail_hbm, wp_hbm, ibuf, obuf, isem, osem):
        wid = lax.axis_index("s") * _NUM_CORES + lax.axis_index("c")
        iota = lax.iota(jnp.int32, _L)
        qrow = [iota // 2 + q0 for q0 in (0, 8, 16, 24, 32, 40, 48, 56)]
        scol = [iota + 16 * qi for qi in range(8)]
        e64 = (iota % 2) * 64

        def in_start(c, b):
            pltpu.async_copy(
                wt_hbm.at[:, pl.ds(c * 128, 128)], ibuf.at[b], isem.at[b])

        def in_wait(c, b):
            pltpu.make_async_copy(
                wt_hbm.at[:, pl.ds(c * 128, 128)], ibuf.at[b],
                isem.at[b]).wait()

        def out_start(c, b):
            pltpu.async_copy(
                obuf.at[b], wp_hbm.at[pl.ds(c * 64, 64)], osem.at[b])

        def out_wait(c, b):
            pltpu.make_async_copy(
                obuf.at[b], wp_hbm.at[pl.ds(c * 64, 64)], osem.at[b]).wait()

        def shuffle(src, dst, nq):
            def jstep(jb, carry):
                for dj in range(8):
                    jm = (iota + jb * 8 + dj) & (D - 1)
                    dcol = e64 + jm
                    for qi in range(nq // 8):
                        v = plsc.load_gather(src, [jm, scol[qi]])
                        plsc.store_scatter(dst, [qrow[qi], dcol], v)
                return carry
            lax.fori_loop(0, D // 8, jstep, 0)

        for b in range(_NP):
            in_start(wid + _NW * b, b)

        def step(k3, carry):
            for b in range(_NP):
                k = _NP * k3 + b
                c = wid + _NW * k

                @pl.when(c < n_full)
                def _():
                    in_wait(c, b)

                @pl.when((k >= _NP) & (c - _NP * _NW < n_full))
                def _():
                    out_wait(c - _NP * _NW, b)

                @pl.when(c < n_full)
                def _():
                    shuffle(ibuf.at[b], obuf.at[b], D)

                    @pl.when(c + _NP * _NW < n_full)
                    def _():
                        in_start(c + _NP * _NW, b)

                    out_start(c, b)
            return carry

        lax.fori_loop(0, (n_steps + _NP - 1) // _NP, step, 0)

        @pl.when(wid == _NW - 1)
        def _():
            pltpu.sync_copy(tail_hbm, ibuf.at[0])
            shuffle(ibuf.at[0], obuf.at[0], rem // 2)
            pltpu.sync_copy(
                obuf.at[0, pl.ds(0, rem // 2)],
                wp_hbm.at[pl.ds(n_full * 64, rem // 2)])

    return body


def _make_gather(S, Bt, V, D, n_per_w):
    n_chunks_b = Bt // _G
    mesh = plsc.VectorSubcoreMesh(core_axis_name="c", subcore_axis_name="s")

    @functools.partial(
        pl.kernel,
        mesh=mesh,
        out_type=jax.ShapeDtypeStruct((S, D, Bt), jnp.float32),
        scratch_types=[
            pltpu.VMEM((n_per_w, _G), jnp.int32),
            pltpu.VMEM((_NG, _G), jnp.int32),
            pltpu.VMEM((_NG, _G, 2 * D), jnp.float32),
            pltpu.VMEM((_NO, D, _G), jnp.float32),
            pltpu.SemaphoreType.DMA((_NG,)),
            pltpu.SemaphoreType.DMA((_NO,)),
        ],
        compiler_params=_CPARAMS,
    )
    def body(idx_hbm, wp_hbm, out_hbm, rawv, pbuf, gbuf, obuf, gsem, osem):
        wid = lax.axis_index("s") * _NUM_CORES + lax.axis_index("c")
        cid0 = wid * n_per_w
        pltpu.sync_copy(idx_hbm.at[pl.ds(cid0, n_per_w)], rawv)

        iota = lax.iota(jnp.int32, _L)
        tvec = [iota + _L * tg for tg in range(_G // _L)]

        def make_plist(g, b):
            for tg in range(_G // _L):
                pbuf[b, pl.ds(tg * _L, _L)] = (
                    rawv[g, pl.ds(tg * _L, _L)] >> 1)

        def gather_start(g, b):
            pltpu.async_copy(wp_hbm.at[pbuf.at[b]], gbuf.at[b], gsem.at[b])

        def gather_wait(g, b):
            pltpu.make_async_copy(
                wp_hbm.at[pbuf.at[b]], gbuf.at[b], gsem.at[b]).wait()

        def out_ref(g):
            cid = cid0 + g
            s = cid // n_chunks_b
            c = cid % n_chunks_b
            return out_hbm.at[s, :, pl.ds(c * _G, _G)]

        def store_start(g, b):
            pltpu.async_copy(obuf.at[b], out_ref(g), osem.at[b])

        def store_wait(g, b):
            pltpu.make_async_copy(obuf.at[b], out_ref(g), osem.at[b]).wait()

        for b in range(_NG):
            make_plist(b, b)
            gather_start(b, b)

        def step(g, carry):
            b = g % _NG
            b2 = g % _NO
            gather_wait(g, b)

            @pl.when(g >= _NO)
            def _():
                store_wait(g - _NO, b2)

            gb = gbuf.at[b]
            ob = obuf.at[b2]
            parcol = [(rawv[g, pl.ds(tg * _L, _L)] & 1) * D
                      for tg in range(_G // _L)]

            def jstep(jb, carry):
                for dj in range(8):
                    jm = (iota + jb * 8 + dj) & (D - 1)
                    for tg in range(_G // _L):
                        v = plsc.load_gather(
                            gb, [tvec[tg], parcol[tg] + jm])
                        plsc.store_scatter(ob, [jm, tvec[tg]], v)
                return carry
            lax.fori_loop(0, D // 8, jstep, 0)

            @pl.when(g + _NG < n_per_w)
            def _():
                make_plist(g + _NG, b)
                gather_start(g + _NG, b)

            store_start(g, b2)
            return carry

        lax.fori_loop(0, n_per_w, step, 0)

        for j in range(_NO):
            g = n_per_w - _NO + j
            store_wait(g, g % _NO)

    return body


def kernel(token_ids, weight):
    Bt, S = token_ids.shape
    V, D = weight.shape
    n_chunks = (Bt * S) // _G
    n_per_w = n_chunks // _NW
    n_full = V // 128
    rem = V - n_full * 128

    wt = weight.T
    tail = jnp.pad(weight[n_full * 128:].T, ((0, 0), (0, 128 - rem)))
    wp = _make_pack(V, D)(wt, tail)
    idx = token_ids.T.astype(jnp.int32).reshape(n_chunks, _G)
    out_t = _make_gather(S, Bt, V, D, n_per_w)(idx, wp)
    return jnp.transpose(out_t, (2, 0, 1))


# speedup vs baseline: 2.2477x; 2.2477x over previous
import functools

import jax
import jax.numpy as jnp
from jax import lax
from jax.experimental import pallas as pl
from jax.experimental.pallas import tpu as pltpu
from jax.experimental.pallas import tpu_sc as plsc

_NUM_CORES = 2
_NUM_SUBCORES = 16
_NW = _NUM_CORES * _NUM_SUBCORES
_G = 128
_NG = 3
_NO = 2
_NP = 3
_L = 16

_CPARAMS = pltpu.CompilerParams(needs_layout_passes=False)


def _make_pack(V, D):
    n_full = V // 128
    rem = V - n_full * 128
    mesh = plsc.VectorSubcoreMesh(core_axis_name="c", subcore_axis_name="s")
    kmax = (n_full + _NW - 1) // _NW
    n_steps = kmax + _NP

    @functools.partial(
        pl.kernel,
        mesh=mesh,
        out_type=jax.ShapeDtypeStruct((V // 2, 2 * D), jnp.float32),
        scratch_types=[
            pltpu.VMEM((_NP, D, 2 * D), jnp.float32),
            pltpu.VMEM((_NP, D, 2 * D), jnp.float32),
            pltpu.SemaphoreType.DMA((_NP,)),
            pltpu.SemaphoreType.DMA((_NP,)),
        ],
        compiler_params=_CPARAMS,
    )
    def body(wt_hbm, tail_hbm, wp_hbm, ibuf, obuf, isem, osem):
        wid = lax.axis_index("s") * _NUM_CORES + lax.axis_index("c")
        iota = lax.iota(jnp.int32, _L)
        qrow = [iota // 2 + q0 for q0 in (0, 8, 16, 24, 32, 40, 48, 56)]
        scol = [iota + 16 * qi for qi in range(8)]
        e64 = (iota % 2) * 64

        def in_start(c, b):
            pltpu.async_copy(
                wt_hbm.at[:, pl.ds(c * 128, 128)], ibuf.at[b], isem.at[b])

        def in_wait(c, b):
            pltpu.make_async_copy(
                wt_hbm.at[:, pl.ds(c * 128, 128)], ibuf.at[b],
                isem.at[b]).wait()

        def out_start(c, b):
            pltpu.async_copy(
                obuf.at[b], wp_hbm.at[pl.ds(c * 64, 64)], osem.at[b])

        def out_wait(c, b):
            pltpu.make_async_copy(
                obuf.at[b], wp_hbm.at[pl.ds(c * 64, 64)], osem.at[b]).wait()

        def shuffle(src, dst, nq):
            def jstep(jb, carry):
                for dj in range(8):
                    jm = (iota + jb * 8 + dj) & (D - 1)
                    dcol = e64 + jm
                    vs = [plsc.load_gather(src, [jm, scol[qi]])
                          for qi in range(nq // 8)]
                    for qi in range(nq // 8):
                        plsc.store_scatter(dst, [qrow[qi], dcol], vs[qi])
                return carry
            lax.fori_loop(0, D // 8, jstep, 0)

        for b in range(_NP):
            in_start(wid + _NW * b, b)

        def step(k3, carry):
            for b in range(_NP):
                k = _NP * k3 + b
                c = wid + _NW * k

                @pl.when(c < n_full)
                def _():
                    in_wait(c, b)

                @pl.when((k >= _NP) & (c - _NP * _NW < n_full))
                def _():
                    out_wait(c - _NP * _NW, b)

                @pl.when(c < n_full)
                def _():
                    shuffle(ibuf.at[b], obuf.at[b], D)

                    @pl.when(c + _NP * _NW < n_full)
                    def _():
                        in_start(c + _NP * _NW, b)

                    out_start(c, b)
            return carry

        lax.fori_loop(0, (n_steps + _NP - 1) // _NP, step, 0)

        @pl.when(wid == _NW - 1)
        def _():
            pltpu.sync_copy(tail_hbm, ibuf.at[0])
            shuffle(ibuf.at[0], obuf.at[0], rem // 2)
            pltpu.sync_copy(
                obuf.at[0, pl.ds(0, rem // 2)],
                wp_hbm.at[pl.ds(n_full * 64, rem // 2)])

    return body


def _make_gather(S, Bt, V, D, n_per_w):
    n_chunks_b = Bt // _G
    mesh = plsc.VectorSubcoreMesh(core_axis_name="c", subcore_axis_name="s")

    @functools.partial(
        pl.kernel,
        mesh=mesh,
        out_type=jax.ShapeDtypeStruct((S, D, Bt), jnp.float32),
        scratch_types=[
            pltpu.VMEM((n_per_w, _G), jnp.int32),
            pltpu.VMEM((_NG, _G), jnp.int32),
            pltpu.VMEM((_NG, _G, 2 * D), jnp.float32),
            pltpu.VMEM((_NO, D, _G), jnp.float32),
            pltpu.SemaphoreType.DMA((_NG,)),
            pltpu.SemaphoreType.DMA((_NO,)),
        ],
        compiler_params=_CPARAMS,
    )
    def body(idx_hbm, wp_hbm, out_hbm, rawv, pbuf, gbuf, obuf, gsem, osem):
        wid = lax.axis_index("s") * _NUM_CORES + lax.axis_index("c")
        cid0 = wid * n_per_w
        pltpu.sync_copy(idx_hbm.at[pl.ds(cid0, n_per_w)], rawv)

        iota = lax.iota(jnp.int32, _L)
        tvec = [iota + _L * tg for tg in range(_G // _L)]

        def make_plist(g, b):
            for tg in range(_G // _L):
                pbuf[b, pl.ds(tg * _L, _L)] = (
                    rawv[g, pl.ds(tg * _L, _L)] >> 1)

        def gather_start(g, b):
            pltpu.async_copy(wp_hbm.at[pbuf.at[b]], gbuf.at[b], gsem.at[b])

        def gather_wait(g, b):
            pltpu.make_async_copy(
                wp_hbm.at[pbuf.at[b]], gbuf.at[b], gsem.at[b]).wait()

        def out_ref(g):
            cid = cid0 + g
            s = cid // n_chunks_b
            c = cid % n_chunks_b
            return out_hbm.at[s, :, pl.ds(c * _G, _G)]

        def store_start(g, b):
            pltpu.async_copy(obuf.at[b], out_ref(g), osem.at[b])

        def store_wait(g, b):
            pltpu.make_async_copy(obuf.at[b], out_ref(g), osem.at[b]).wait()

        for b in range(_NG):
            make_plist(b, b)
            gather_start(b, b)

        def step(g, carry):
            b = g % _NG
            b2 = g % _NO
            gather_wait(g, b)

            @pl.when(g >= _NO)
            def _():
                store_wait(g - _NO, b2)

            gb = gbuf.at[b]
            ob = obuf.at[b2]
            parcol = [(rawv[g, pl.ds(tg * _L, _L)] & 1) * D
                      for tg in range(_G // _L)]

            def jstep(jb, carry):
                for dj in range(8):
                    jm = (iota + jb * 8 + dj) & (D - 1)
                    vs = [plsc.load_gather(gb, [tvec[tg], parcol[tg] + jm])
                          for tg in range(_G // _L)]
                    for tg in range(_G // _L):
                        plsc.store_scatter(ob, [jm, tvec[tg]], vs[tg])
                return carry
            lax.fori_loop(0, D // 8, jstep, 0)

            @pl.when(g + _NG < n_per_w)
            def _():
                make_plist(g + _NG, b)
                gather_start(g + _NG, b)

            store_start(g, b2)
            return carry

        lax.fori_loop(0, n_per_w, step, 0)

        for j in range(_NO):
            g = n_per_w - _NO + j
            store_wait(g, g % _NO)

    return body


def kernel(token_ids, weight):
    Bt, S = token_ids.shape
    V, D = weight.shape
    n_chunks = (Bt * S) // _G
    n_per_w = n_chunks // _NW
    n_full = V // 128
    rem = V - n_full * 128

    wt = weight.T
    tail = jnp.pad(weight[n_full * 128:].T, ((0, 0), (0, 128 - rem)))
    wp = _make_pack(V, D)(wt, tail)
    idx = token_ids.T.astype(jnp.int32).reshape(n_chunks, _G)
    out_t = _make_gather(S, Bt, V, D, n_per_w)(idx, wp)
    return jnp.transpose(out_t, (2, 0, 1))


# speedup vs baseline: 2.2493x; 1.0007x over previous
import functools

import jax
import jax.numpy as jnp
from jax import lax
from jax.experimental import pallas as pl
from jax.experimental.pallas import tpu as pltpu
from jax.experimental.pallas import tpu_sc as plsc

_NUM_CORES = 2
_NUM_SUBCORES = 16
_NW = _NUM_CORES * _NUM_SUBCORES
_G = 128
_NG = 3
_NO = 2
_NP = 3
_L = 16

_CPARAMS = pltpu.CompilerParams(needs_layout_passes=False)


def _make_pack(V, D):
    n_full = V // 128
    rem = V - n_full * 128
    mesh = plsc.VectorSubcoreMesh(core_axis_name="c", subcore_axis_name="s")
    kmax = (n_full + _NW - 1) // _NW
    n_steps = kmax + _NP

    @functools.partial(
        pl.kernel,
        mesh=mesh,
        out_type=jax.ShapeDtypeStruct((V // 2, 2 * D), jnp.float32),
        scratch_types=[
            pltpu.VMEM((_NP, D, 2 * D), jnp.float32),
            pltpu.VMEM((_NP, D, 2 * D), jnp.float32),
            pltpu.SemaphoreType.DMA((_NP,)),
            pltpu.SemaphoreType.DMA((_NP,)),
        ],
        compiler_params=_CPARAMS,
    )
    def body(wt_hbm, tail_hbm, wp_hbm, ibuf, obuf, isem, osem):
        wid = lax.axis_index("s") * _NUM_CORES + lax.axis_index("c")
        iota = lax.iota(jnp.int32, _L)
        qrow = [iota // 2 + q0 for q0 in (0, 8, 16, 24, 32, 40, 48, 56)]
        scol = [iota + 16 * qi for qi in range(8)]
        e64 = (iota % 2) * 64

        def in_start(c, b):
            pltpu.async_copy(
                wt_hbm.at[:, pl.ds(c * 128, 128)], ibuf.at[b], isem.at[b])

        def in_wait(c, b):
            pltpu.make_async_copy(
                wt_hbm.at[:, pl.ds(c * 128, 128)], ibuf.at[b],
                isem.at[b]).wait()

        def out_start(c, b):
            pltpu.async_copy(
                obuf.at[b], wp_hbm.at[pl.ds(c * 64, 64)], osem.at[b])

        def out_wait(c, b):
            pltpu.make_async_copy(
                obuf.at[b], wp_hbm.at[pl.ds(c * 64, 64)], osem.at[b]).wait()

        def shuffle(src, dst, nq):
            def jstep(jb, carry):
                def loads(dj):
                    jm = (iota + jb * 8 + dj) & (D - 1)
                    return e64 + jm, [
                        plsc.load_gather(src, [jm, scol[qi]])
                        for qi in range(nq // 8)]

                def stores(dcol, vs):
                    for qi in range(nq // 8):
                        plsc.store_scatter(dst, [qrow[qi], dcol], vs[qi])

                prev = loads(0)
                for dj in range(1, 8):
                    cur = loads(dj)
                    stores(*prev)
                    prev = cur
                stores(*prev)
                return carry
            lax.fori_loop(0, D // 8, jstep, 0)

        for b in range(_NP):
            in_start(wid + _NW * b, b)

        def step(k3, carry):
            for b in range(_NP):
                k = _NP * k3 + b
                c = wid + _NW * k

                @pl.when(c < n_full)
                def _():
                    in_wait(c, b)

                @pl.when((k >= _NP) & (c - _NP * _NW < n_full))
                def _():
                    out_wait(c - _NP * _NW, b)

                @pl.when(c < n_full)
                def _():
                    shuffle(ibuf.at[b], obuf.at[b], D)

                    @pl.when(c + _NP * _NW < n_full)
                    def _():
                        in_start(c + _NP * _NW, b)

                    out_start(c, b)
            return carry

        lax.fori_loop(0, (n_steps + _NP - 1) // _NP, step, 0)

        @pl.when(wid == _NW - 1)
        def _():
            pltpu.sync_copy(tail_hbm, ibuf.at[0])
            shuffle(ibuf.at[0], obuf.at[0], rem // 2)
            pltpu.sync_copy(
                obuf.at[0, pl.ds(0, rem // 2)],
                wp_hbm.at[pl.ds(n_full * 64, rem // 2)])

    return body


def _make_gather(S, Bt, V, D, n_per_w):
    n_chunks_b = Bt // _G
    mesh = plsc.VectorSubcoreMesh(core_axis_name="c", subcore_axis_name="s")

    @functools.partial(
        pl.kernel,
        mesh=mesh,
        out_type=jax.ShapeDtypeStruct((S, D, Bt), jnp.float32),
        scratch_types=[
            pltpu.VMEM((n_per_w, _G), jnp.int32),
            pltpu.VMEM((_NG, _G), jnp.int32),
            pltpu.VMEM((_NG, _G, 2 * D), jnp.float32),
            pltpu.VMEM((_NO, D, _G), jnp.float32),
            pltpu.SemaphoreType.DMA((_NG,)),
            pltpu.SemaphoreType.DMA((_NO,)),
        ],
        compiler_params=_CPARAMS,
    )
    def body(idx_hbm, wp_hbm, out_hbm, rawv, pbuf, gbuf, obuf, gsem, osem):
        wid = lax.axis_index("s") * _NUM_CORES + lax.axis_index("c")
        cid0 = wid * n_per_w
        pltpu.sync_copy(idx_hbm.at[pl.ds(cid0, n_per_w)], rawv)

        iota = lax.iota(jnp.int32, _L)
        tvec = [iota + _L * tg for tg in range(_G // _L)]

        def make_plist(g, b):
            for tg in range(_G // _L):
                pbuf[b, pl.ds(tg * _L, _L)] = (
                    rawv[g, pl.ds(tg * _L, _L)] >> 1)

        def gather_start(g, b):
            pltpu.async_copy(wp_hbm.at[pbuf.at[b]], gbuf.at[b], gsem.at[b])

        def gather_wait(g, b):
            pltpu.make_async_copy(
                wp_hbm.at[pbuf.at[b]], gbuf.at[b], gsem.at[b]).wait()

        def out_ref(g):
            cid = cid0 + g
            s = cid // n_chunks_b
            c = cid % n_chunks_b
            return out_hbm.at[s, :, pl.ds(c * _G, _G)]

        def store_start(g, b):
            pltpu.async_copy(obuf.at[b], out_ref(g), osem.at[b])

        def store_wait(g, b):
            pltpu.make_async_copy(obuf.at[b], out_ref(g), osem.at[b]).wait()

        for b in range(_NG):
            make_plist(b, b)
            gather_start(b, b)

        def step(g, carry):
            b = g % _NG
            b2 = g % _NO
            gather_wait(g, b)

            @pl.when(g >= _NO)
            def _():
                store_wait(g - _NO, b2)

            gb = gbuf.at[b]
            ob = obuf.at[b2]
            parcol = [(rawv[g, pl.ds(tg * _L, _L)] & 1) * D
                      for tg in range(_G // _L)]

            def jstep(jb, carry):
                def loads(dj):
                    jm = (iota + jb * 8 + dj) & (D - 1)
                    return jm, [
                        plsc.load_gather(gb, [tvec[tg], parcol[tg] + jm])
                        for tg in range(_G // _L)]

                def stores(jm, vs):
                    for tg in range(_G // _L):
                        plsc.store_scatter(ob, [jm, tvec[tg]], vs[tg])

                prev = loads(0)
                for dj in range(1, 8):
                    cur = loads(dj)
                    stores(*prev)
                    prev = cur
                stores(*prev)
                return carry
            lax.fori_loop(0, D // 8, jstep, 0)

            @pl.when(g + _NG < n_per_w)
            def _():
                make_plist(g + _NG, b)
                gather_start(g + _NG, b)

            store_start(g, b2)
            return carry

        lax.fori_loop(0, n_per_w, step, 0)

        for j in range(_NO):
            g = n_per_w - _NO + j
            store_wait(g, g % _NO)

    return body


def kernel(token_ids, weight):
    Bt, S = token_ids.shape
    V, D = weight.shape
    n_chunks = (Bt * S) // _G
    n_per_w = n_chunks // _NW
    n_full = V // 128
    rem = V - n_full * 128

    wt = weight.T
    tail = jnp.pad(weight[n_full * 128:].T, ((0, 0), (0, 128 - rem)))
    wp = _make_pack(V, D)(wt, tail)
    idx = token_ids.T.astype(jnp.int32).reshape(n_chunks, _G)
    out_t = _make_gather(S, Bt, V, D, n_per_w)(idx, wp)
    return jnp.transpose(out_t, (2, 0, 1))


# speedup vs baseline: 2.2622x; 1.0057x over previous
import functools

import jax
import jax.numpy as jnp
from jax import lax
from jax.experimental import pallas as pl
from jax.experimental.pallas import tpu as pltpu
from jax.experimental.pallas import tpu_sc as plsc

_NUM_CORES = 2
_NUM_SUBCORES = 16
_NW = _NUM_CORES * _NUM_SUBCORES
_G = 128
_NG = 4
_NO = 3
_NP = 3
_L = 16

_CPARAMS = pltpu.CompilerParams(needs_layout_passes=False)


def _make_pack(V, D):
    n_full = V // 128
    rem = V - n_full * 128
    mesh = plsc.VectorSubcoreMesh(core_axis_name="c", subcore_axis_name="s")
    kmax = (n_full + _NW - 1) // _NW
    n_steps = kmax + _NP

    @functools.partial(
        pl.kernel,
        mesh=mesh,
        out_type=jax.ShapeDtypeStruct((V // 2, 2 * D), jnp.float32),
        scratch_types=[
            pltpu.VMEM((_NP, D, 2 * D), jnp.float32),
            pltpu.VMEM((_NP, D, 2 * D), jnp.float32),
            pltpu.SemaphoreType.DMA((_NP,)),
            pltpu.SemaphoreType.DMA((_NP,)),
        ],
        compiler_params=_CPARAMS,
    )
    def body(wt_hbm, tail_hbm, wp_hbm, ibuf, obuf, isem, osem):
        wid = lax.axis_index("s") * _NUM_CORES + lax.axis_index("c")
        iota = lax.iota(jnp.int32, _L)
        qrow = [iota // 2 + q0 for q0 in (0, 8, 16, 24, 32, 40, 48, 56)]
        scol = [iota + 16 * qi for qi in range(8)]
        e64 = (iota % 2) * 64

        def in_start(c, b):
            pltpu.async_copy(
                wt_hbm.at[:, pl.ds(c * 128, 128)], ibuf.at[b], isem.at[b])

        def in_wait(c, b):
            pltpu.make_async_copy(
                wt_hbm.at[:, pl.ds(c * 128, 128)], ibuf.at[b],
                isem.at[b]).wait()

        def out_start(c, b):
            pltpu.async_copy(
                obuf.at[b], wp_hbm.at[pl.ds(c * 64, 64)], osem.at[b])

        def out_wait(c, b):
            pltpu.make_async_copy(
                obuf.at[b], wp_hbm.at[pl.ds(c * 64, 64)], osem.at[b]).wait()

        def shuffle(src, dst, nq):
            def jstep(jb, carry):
                def loads(dj):
                    jm = (iota + jb * 8 + dj) & (D - 1)
                    return e64 + jm, [
                        plsc.load_gather(src, [jm, scol[qi]])
                        for qi in range(nq // 8)]

                def stores(dcol, vs):
                    for qi in range(nq // 8):
                        plsc.store_scatter(dst, [qrow[qi], dcol], vs[qi])

                prev = loads(0)
                for dj in range(1, 8):
                    cur = loads(dj)
                    stores(*prev)
                    prev = cur
                stores(*prev)
                return carry
            lax.fori_loop(0, D // 8, jstep, 0)

        for b in range(_NP):
            in_start(wid + _NW * b, b)

        def step(k3, carry):
            for b in range(_NP):
                k = _NP * k3 + b
                c = wid + _NW * k

                @pl.when(c < n_full)
                def _():
                    in_wait(c, b)

                @pl.when((k >= _NP) & (c - _NP * _NW < n_full))
                def _():
                    out_wait(c - _NP * _NW, b)

                @pl.when(c < n_full)
                def _():
                    shuffle(ibuf.at[b], obuf.at[b], D)

                    @pl.when(c + _NP * _NW < n_full)
                    def _():
                        in_start(c + _NP * _NW, b)

                    out_start(c, b)
            return carry

        lax.fori_loop(0, (n_steps + _NP - 1) // _NP, step, 0)

        @pl.when(wid == _NW - 1)
        def _():
            pltpu.sync_copy(tail_hbm, ibuf.at[0])
            shuffle(ibuf.at[0], obuf.at[0], rem // 2)
            pltpu.sync_copy(
                obuf.at[0, pl.ds(0, rem // 2)],
                wp_hbm.at[pl.ds(n_full * 64, rem // 2)])

    return body


def _make_gather(S, Bt, V, D, n_per_w):
    n_chunks_b = Bt // _G
    mesh = plsc.VectorSubcoreMesh(core_axis_name="c", subcore_axis_name="s")

    @functools.partial(
        pl.kernel,
        mesh=mesh,
        out_type=jax.ShapeDtypeStruct((S, D, Bt), jnp.float32),
        scratch_types=[
            pltpu.VMEM((n_per_w, _G), jnp.int32),
            pltpu.VMEM((_NG, _G), jnp.int32),
            pltpu.VMEM((_NG, _G, 2 * D), jnp.float32),
            pltpu.VMEM((_NO, D, _G), jnp.float32),
            pltpu.SemaphoreType.DMA((_NG,)),
            pltpu.SemaphoreType.DMA((_NO,)),
        ],
        compiler_params=_CPARAMS,
    )
    def body(idx_hbm, wp_hbm, out_hbm, rawv, pbuf, gbuf, obuf, gsem, osem):
        wid = lax.axis_index("s") * _NUM_CORES + lax.axis_index("c")
        cid0 = wid * n_per_w
        pltpu.sync_copy(idx_hbm.at[pl.ds(cid0, n_per_w)], rawv)

        iota = lax.iota(jnp.int32, _L)
        tvec = [iota + _L * tg for tg in range(_G // _L)]

        def make_plist(g, b):
            for tg in range(_G // _L):
                pbuf[b, pl.ds(tg * _L, _L)] = (
                    rawv[g, pl.ds(tg * _L, _L)] >> 1)

        def gather_start(g, b):
            pltpu.async_copy(wp_hbm.at[pbuf.at[b]], gbuf.at[b], gsem.at[b])

        def gather_wait(g, b):
            pltpu.make_async_copy(
                wp_hbm.at[pbuf.at[b]], gbuf.at[b], gsem.at[b]).wait()

        def out_ref(g):
            cid = cid0 + g
            s = cid // n_chunks_b
            c = cid % n_chunks_b
            return out_hbm.at[s, :, pl.ds(c * _G, _G)]

        def store_start(g, b):
            pltpu.async_copy(obuf.at[b], out_ref(g), osem.at[b])

        def store_wait(g, b):
            pltpu.make_async_copy(obuf.at[b], out_ref(g), osem.at[b]).wait()

        for b in range(_NG):
            make_plist(b, b)
            gather_start(b, b)

        def step(g, carry):
            b = g % _NG
            b2 = g % _NO
            gather_wait(g, b)

            @pl.when(g >= _NO)
            def _():
                store_wait(g - _NO, b2)

            gb = gbuf.at[b]
            ob = obuf.at[b2]
            parcol = [(rawv[g, pl.ds(tg * _L, _L)] & 1) * D
                      for tg in range(_G // _L)]

            def jstep(jb, carry):
                def loads(dj):
                    jm = (iota + jb * 8 + dj) & (D - 1)
                    return jm, [
                        plsc.load_gather(gb, [tvec[tg], parcol[tg] + jm])
                        for tg in range(_G // _L)]

                def stores(jm, vs):
                    for tg in range(_G // _L):
                        plsc.store_scatter(ob, [jm, tvec[tg]], vs[tg])

                prev = loads(0)
                for dj in range(1, 8):
                    cur = loads(dj)
                    stores(*prev)
                    prev = cur
                stores(*prev)
                return carry
            lax.fori_loop(0, D // 8, jstep, 0)

            @pl.when(g + _NG < n_per_w)
            def _():
                make_plist(g + _NG, b)
                gather_start(g + _NG, b)

            store_start(g, b2)
            return carry

        lax.fori_loop(0, n_per_w, step, 0)

        for j in range(_NO):
            g = n_per_w - _NO + j
            store_wait(g, g % _NO)

    return body


def kernel(token_ids, weight):
    Bt, S = token_ids.shape
    V, D = weight.shape
    n_chunks = (Bt * S) // _G
    n_per_w = n_chunks // _NW
    n_full = V // 128
    rem = V - n_full * 128

    wt = weight.T
    tail = jnp.pad(weight[n_full * 128:].T, ((0, 0), (0, 128 - rem)))
    wp = _make_pack(V, D)(wt, tail)
    idx = token_ids.T.astype(jnp.int32).reshape(n_chunks, _G)
    out_t = _make_gather(S, Bt, V, D, n_per_w)(idx, wp)
    return jnp.transpose(out_t, (2, 0, 1))
